# Initial kernel scaffold; baseline (speedup 1.0000x reference)
import jax
import jax.numpy as jnp
from jax.experimental import pallas as pl


def kernel(node_feat, edge_index, W1, b1, gamma, beta, W2, b2):
    raise NotImplementedError("write your pallas kernel here")


# speedup vs baseline: 9.5593x; 9.5593x over previous
import dataclasses
import functools

import jax
import jax.numpy as jnp
from jax import lax
from jax.experimental import pallas as pl
from jax.experimental.pallas import tpu as pltpu
from jax.experimental.pallas import tpu_sc as plsc

_N = 10000
_D = 128
_E = 320000
_NP = 10240
_CHUNK = 128
_NSUB = 16
_NCORE = 2
_NW = _NSUB * _NCORE
_EP = 327680
_EROWS = _EP // _CHUNK
_ROWS_W = _EROWS // _NW
_ACC_W = _NP // _NSUB
_BN_EPS = 1e-5
_BM = 1024
_G = _NP // _BM


def _sc_mesh():
    return plsc.VectorSubcoreMesh(core_axis_name="c", subcore_axis_name="s")


def _sc_compiler_params():
    cp = pltpu.CompilerParams()
    if "needs_layout_passes" in pltpu.CompilerParams.__dataclass_fields__:
        cp = dataclasses.replace(cp, needs_layout_passes=False)
    return cp


def _sc_degree(dst_rows):
    @functools.partial(
        pl.kernel,
        out_type=jax.ShapeDtypeStruct((_NW, _NP), jnp.float32),
        mesh=_sc_mesh(),
        compiler_params=_sc_compiler_params(),
        scratch_types=[
            pltpu.VMEM((_ROWS_W, _CHUNK), jnp.int32),
            pltpu.VMEM((_NP,), jnp.float32),
            pltpu.SemaphoreType.DMA,
            pltpu.SemaphoreType.DMA,
        ],
    )
    def k(dst_hbm, out_hbm, didx, hist, s0, s1):
        c = lax.axis_index("c")
        s = lax.axis_index("s")
        wid = c * _NSUB + s
        pltpu.async_copy(dst_hbm.at[pl.ds(wid * _ROWS_W, _ROWS_W)], didx,
                         s0).wait()

        @pl.loop(0, _NP // 16)
        def _(i):
            hist[pl.ds(i * 16, 16)] = jnp.zeros((16,), jnp.float32)

        @pl.loop(0, _ROWS_W)
        def _(j):
            @pl.loop(0, _CHUNK // 16)
            def _(kk):
                iv = didx[j, pl.ds(kk * 16, 16)]
                plsc.addupdate_scatter(hist, [iv],
                                       jnp.ones((16,), jnp.float32))

        pltpu.async_copy(hist, out_hbm.at[wid], s1).wait()

    return k(dst_rows)


def _tc_deg_reduce(deg_parts):
    def body(d_ref, o_ref):
        o_ref[...] = lax.rsqrt(
            jnp.sum(d_ref[...], axis=0, keepdims=True) + 1.0)

    return pl.pallas_call(
        body,
        grid=(_G,),
        in_specs=[pl.BlockSpec((_NW, _BM), lambda i: (0, i))],
        out_specs=pl.BlockSpec((1, _BM), lambda i: (0, i)),
        out_shape=jax.ShapeDtypeStruct((1, _NP), jnp.float32),
    )(deg_parts)


def _sc_aggregate(hp, src_rows, dst_rows):
    @functools.partial(
        pl.kernel,
        out_type=jax.ShapeDtypeStruct((2 * _NP, _D), jnp.float32),
        mesh=_sc_mesh(),
        scratch_types=[
            pltpu.VMEM((_ROWS_W, _CHUNK), jnp.int32),
            pltpu.VMEM((_ROWS_W, _CHUNK), jnp.int32),
            pltpu.VMEM((_CHUNK, _D), jnp.float32),
            pltpu.VMEM_SHARED((_NP, _D), jnp.float32),
        ],
    )
    def k(hp_hbm, src_hbm, dst_hbm, out_hbm, sidx, didx, buf, acc):
        c = lax.axis_index("c")
        s = lax.axis_index("s")
        wid = c * _NSUB + s
        pltpu.sync_copy(src_hbm.at[pl.ds(wid * _ROWS_W, _ROWS_W)], sidx)
        pltpu.sync_copy(dst_hbm.at[pl.ds(wid * _ROWS_W, _ROWS_W)], didx)
        r0 = s * _ACC_W
        pltpu.sync_copy(hp_hbm.at[pl.ds(r0, _ACC_W)],
                        acc.at[pl.ds(r0, _ACC_W)])
        plsc.subcore_barrier()

        @pl.loop(0, _ROWS_W)
        def _(j):
            pltpu.sync_copy(hp_hbm.at[sidx.at[j]], buf)
            pltpu.sync_copy(buf, acc.at[didx.at[j]], add=True)

        plsc.subcore_barrier()
        pltpu.sync_copy(acc.at[pl.ds(r0, _ACC_W)],
                        out_hbm.at[pl.ds(c * _NP + r0, _ACC_W)])

    return k(hp, src_rows, dst_rows)


def _tc_matmul(x, w):
    def body(x_ref, w_ref, o_ref):
        o_ref[...] = jnp.dot(x_ref[...], w_ref[...],
                             preferred_element_type=jnp.float32)

    return pl.pallas_call(
        body,
        grid=(_G,),
        in_specs=[pl.BlockSpec((_BM, _D), lambda i: (i, 0)),
                  pl.BlockSpec((_D, _D), lambda i: (0, 0))],
        out_specs=pl.BlockSpec((_BM, _D), lambda i: (i, 0)),
        out_shape=jax.ShapeDtypeStruct((_NP, _D), jnp.float32),
    )(x, w)


def _tc_scale(h1, dis):
    def body(h_ref, dis_ref, hp_ref):
        hp_ref[...] = h_ref[...] * dis_ref[...]

    return pl.pallas_call(
        body,
        grid=(_G,),
        in_specs=[pl.BlockSpec((_BM, _D), lambda i: (i, 0)),
                  pl.BlockSpec((_BM, 1), lambda i: (i, 0))],
        out_specs=pl.BlockSpec((_BM, _D), lambda i: (i, 0)),
        out_shape=jax.ShapeDtypeStruct((_NP, _D), jnp.float32),
    )(h1, dis)


def _tc_combine_stats(parts, hp, dis, b):
    def body(p0_ref, p1_ref, hp_ref, dis_ref, b_ref, conv_ref, stats_ref,
             acc_ref):
        i = pl.program_id(0)

        @pl.when(i == 0)
        def _():
            acc_ref[...] = jnp.zeros_like(acc_ref)

        conv = (p0_ref[...] + p1_ref[...] - hp_ref[...]) * dis_ref[...] \
            + b_ref[...]
        conv_ref[...] = conv
        rows = i * _BM + lax.broadcasted_iota(jnp.int32, (_BM, 1), 0)
        cm = jnp.where(rows < _N, conv, 0.0)
        acc_ref[0:1, :] += jnp.sum(cm, axis=0, keepdims=True)
        acc_ref[1:2, :] += jnp.sum(cm * conv, axis=0, keepdims=True)

        @pl.when(i == _G - 1)
        def _():
            stats_ref[...] = acc_ref[...]

    return pl.pallas_call(
        body,
        grid=(_G,),
        in_specs=[pl.BlockSpec((_BM, _D), lambda i: (i, 0)),
                  pl.BlockSpec((_BM, _D), lambda i: (i + _G, 0)),
                  pl.BlockSpec((_BM, _D), lambda i: (i, 0)),
                  pl.BlockSpec((_BM, 1), lambda i: (i, 0)),
                  pl.BlockSpec((1, _D), lambda i: (0, 0))],
        out_specs=[pl.BlockSpec((_BM, _D), lambda i: (i, 0)),
                   pl.BlockSpec((2, _D), lambda i: (0, 0))],
        out_shape=[jax.ShapeDtypeStruct((_NP, _D), jnp.float32),
                   jax.ShapeDtypeStruct((2, _D), jnp.float32)],
        scratch_shapes=[pltpu.VMEM((2, _D), jnp.float32)],
    )(parts, parts, hp, dis, b)


def _tc_bn_matmul(conv, stats, gamma, beta, w2, dis):
    def body(conv_ref, stats_ref, g_ref, be_ref, w_ref, dis_ref, o_ref):
        mean = stats_ref[0:1, :] * (1.0 / _N)
        var = stats_ref[1:2, :] * (1.0 / _N) - mean * mean
        istd = lax.rsqrt(var + _BN_EPS)
        y = (conv_ref[...] - mean) * (istd * g_ref[...]) + be_ref[...]
        y = jnp.maximum(y, 0.0)
        h2 = jnp.dot(y, w_ref[...], preferred_element_type=jnp.float32)
        o_ref[...] = h2 * dis_ref[...]

    return pl.pallas_call(
        body,
        grid=(_G,),
        in_specs=[pl.BlockSpec((_BM, _D), lambda i: (i, 0)),
                  pl.BlockSpec((2, _D), lambda i: (0, 0)),
                  pl.BlockSpec((1, _D), lambda i: (0, 0)),
                  pl.BlockSpec((1, _D), lambda i: (0, 0)),
                  pl.BlockSpec((_D, _D), lambda i: (0, 0)),
                  pl.BlockSpec((_BM, 1), lambda i: (i, 0))],
        out_specs=pl.BlockSpec((_BM, _D), lambda i: (i, 0)),
        out_shape=jax.ShapeDtypeStruct((_NP, _D), jnp.float32),
    )(conv, stats, gamma, beta, w2, dis)


def _tc_final(parts, hp, dis, b):
    def body(p0_ref, p1_ref, hp_ref, dis_ref, b_ref, o_ref):
        o_ref[...] = (p0_ref[...] + p1_ref[...] - hp_ref[...]) \
            * dis_ref[...] + b_ref[...]

    return pl.pallas_call(
        body,
        grid=(_G,),
        in_specs=[pl.BlockSpec((_BM, _D), lambda i: (i, 0)),
                  pl.BlockSpec((_BM, _D), lambda i: (i + _G, 0)),
                  pl.BlockSpec((_BM, _D), lambda i: (i, 0)),
                  pl.BlockSpec((_BM, 1), lambda i: (i, 0)),
                  pl.BlockSpec((1, _D), lambda i: (0, 0))],
        out_specs=pl.BlockSpec((_BM, _D), lambda i: (i, 0)),
        out_shape=jax.ShapeDtypeStruct((_NP, _D), jnp.float32),
    )(parts, parts, hp, dis, b)


def kernel(node_feat, edge_index, W1, b1, gamma, beta, W2, b2):
    src = edge_index[0]
    dst = edge_index[1]
    pad = jnp.full((_EP - _E,), _N, jnp.int32)
    src_rows = jnp.concatenate([src, pad]).reshape(_EROWS, _CHUNK)
    dst_rows = jnp.concatenate([dst, pad]).reshape(_EROWS, _CHUNK)
    x_pad = jnp.zeros((_NP, _D), jnp.float32).at[:_N].set(node_feat)
    b1r = b1.reshape(1, _D)
    b2r = b2.reshape(1, _D)
    gr = gamma.reshape(1, _D)
    ber = beta.reshape(1, _D)

    h1 = _tc_matmul(x_pad, W1)
    degp = _sc_degree(dst_rows)
    dis = _tc_deg_reduce(degp).reshape(_NP, 1)
    h1p = _tc_scale(h1, dis)
    parts1 = _sc_aggregate(h1p, src_rows, dst_rows)
    conv1, stats = _tc_combine_stats(parts1, h1p, dis, b1r)
    h2p = _tc_bn_matmul(conv1, stats, gr, ber, W2, dis)
    parts2 = _sc_aggregate(h2p, src_rows, dst_rows)
    out = _tc_final(parts2, h2p, dis, b2r)
    return out[:_N]


# speedup vs baseline: 10.9499x; 1.1455x over previous
import dataclasses
import functools

import jax
import jax.numpy as jnp
from jax import lax
from jax.experimental import pallas as pl
from jax.experimental.pallas import tpu as pltpu
from jax.experimental.pallas import tpu_sc as plsc

_N = 10000
_D = 128
_E = 320000
_NP = 10240
_CHUNK = 128
_NSUB = 16
_NCORE = 2
_NW = _NSUB * _NCORE
_EP = 327680
_EROWS = _EP // _CHUNK
_ROWS_W = _EROWS // _NW
_ACC_W = _NP // _NSUB
_BN_EPS = 1e-5
_BM = 1024
_G = _NP // _BM


def _sc_mesh():
    return plsc.VectorSubcoreMesh(core_axis_name="c", subcore_axis_name="s")


def _sc_compiler_params():
    cp = pltpu.CompilerParams()
    if "needs_layout_passes" in pltpu.CompilerParams.__dataclass_fields__:
        cp = dataclasses.replace(cp, needs_layout_passes=False)
    return cp


def _sc_degree(dst_rows):
    @functools.partial(
        pl.kernel,
        out_type=jax.ShapeDtypeStruct((_NW, _NP), jnp.float32),
        mesh=_sc_mesh(),
        compiler_params=_sc_compiler_params(),
        scratch_types=[
            pltpu.VMEM((_ROWS_W, _CHUNK), jnp.int32),
            pltpu.VMEM((_NP,), jnp.float32),
            pltpu.SemaphoreType.DMA,
            pltpu.SemaphoreType.DMA,
        ],
    )
    def k(dst_hbm, out_hbm, didx, hist, s0, s1):
        c = lax.axis_index("c")
        s = lax.axis_index("s")
        wid = c * _NSUB + s
        pltpu.async_copy(dst_hbm.at[pl.ds(wid * _ROWS_W, _ROWS_W)], didx,
                         s0).wait()

        @pl.loop(0, _NP // 16)
        def _(i):
            hist[pl.ds(i * 16, 16)] = jnp.zeros((16,), jnp.float32)

        @pl.loop(0, _ROWS_W)
        def _(j):
            @pl.loop(0, _CHUNK // 16)
            def _(kk):
                iv = didx[j, pl.ds(kk * 16, 16)]
                plsc.addupdate_scatter(hist, [iv],
                                       jnp.ones((16,), jnp.float32))

        pltpu.async_copy(hist, out_hbm.at[wid], s1).wait()

    return k(dst_rows)


def _tc_deg_reduce(deg_parts):
    def body(d_ref, o_ref):
        o_ref[...] = lax.rsqrt(
            jnp.sum(d_ref[...], axis=0, keepdims=True) + 1.0)

    return pl.pallas_call(
        body,
        grid=(_G,),
        in_specs=[pl.BlockSpec((_NW, _BM), lambda i: (0, i))],
        out_specs=pl.BlockSpec((1, _BM), lambda i: (0, i)),
        out_shape=jax.ShapeDtypeStruct((1, _NP), jnp.float32),
    )(deg_parts)


def _sc_aggregate(hp, src_rows, dst_rows):
    nbuf = 2
    nphase = 2
    phr = _ROWS_W // nphase

    @functools.partial(
        pl.kernel,
        out_type=jax.ShapeDtypeStruct((2 * _NP, _D), jnp.float32),
        mesh=_sc_mesh(),
        scratch_types=[
            pltpu.VMEM((phr, _CHUNK), jnp.int32),
            pltpu.VMEM((phr, _CHUNK), jnp.int32),
            pltpu.VMEM((_CHUNK, _D), jnp.float32),
            pltpu.VMEM((_CHUNK, _D), jnp.float32),
            pltpu.VMEM_SHARED((_NP, _D), jnp.float32),
            pltpu.SemaphoreType.DMA,
            pltpu.SemaphoreType.DMA,
        ],
    )
    def k(hp_hbm, src_hbm, dst_hbm, out_hbm, sidx, didx,
          buf0, buf1, acc, sm0, sm1):
        c = lax.axis_index("c")
        s = lax.axis_index("s")
        wid = c * _NSUB + s
        row0 = wid * _ROWS_W
        r0 = s * _ACC_W
        pltpu.sync_copy(hp_hbm.at[pl.ds(r0, _ACC_W)],
                        acc.at[pl.ds(r0, _ACC_W)])
        plsc.subcore_barrier()

        bufs = (buf0, buf1)
        sems = (sm0, sm1)
        for ph in range(nphase):
            pltpu.sync_copy(
                src_hbm.at[pl.ds(row0 + ph * phr, phr)], sidx)
            pltpu.sync_copy(
                dst_hbm.at[pl.ds(row0 + ph * phr, phr)], didx)
            for b in range(nbuf):
                pltpu.async_copy(hp_hbm.at[sidx.at[b]], bufs[b], sems[b])

            @pl.loop(0, phr, step=nbuf)
            def _(j, ph=ph):
                for b in range(nbuf):
                    pltpu.make_async_copy(hp_hbm.at[sidx.at[j + b]],
                                          bufs[b], sems[b]).wait()
                    pltpu.sync_copy(bufs[b], acc.at[didx.at[j + b]],
                                    add=True)

                    @pl.when(j + b + nbuf < phr)
                    def _(b=b, j=j):
                        pltpu.async_copy(hp_hbm.at[sidx.at[j + b + nbuf]],
                                         bufs[b], sems[b])

        plsc.subcore_barrier()
        pltpu.sync_copy(acc.at[pl.ds(r0, _ACC_W)],
                        out_hbm.at[pl.ds(c * _NP + r0, _ACC_W)])

    return k(hp, src_rows, dst_rows)


def _tc_matmul(x, w):
    def body(x_ref, w_ref, o_ref):
        o_ref[...] = jnp.dot(x_ref[...], w_ref[...],
                             preferred_element_type=jnp.float32)

    return pl.pallas_call(
        body,
        grid=(_G,),
        in_specs=[pl.BlockSpec((_BM, _D), lambda i: (i, 0)),
                  pl.BlockSpec((_D, _D), lambda i: (0, 0))],
        out_specs=pl.BlockSpec((_BM, _D), lambda i: (i, 0)),
        out_shape=jax.ShapeDtypeStruct((_NP, _D), jnp.float32),
    )(x, w)


def _tc_scale(h1, dis):
    def body(h_ref, dis_ref, hp_ref):
        hp_ref[...] = h_ref[...] * dis_ref[...]

    return pl.pallas_call(
        body,
        grid=(_G,),
        in_specs=[pl.BlockSpec((_BM, _D), lambda i: (i, 0)),
                  pl.BlockSpec((_BM, 1), lambda i: (i, 0))],
        out_specs=pl.BlockSpec((_BM, _D), lambda i: (i, 0)),
        out_shape=jax.ShapeDtypeStruct((_NP, _D), jnp.float32),
    )(h1, dis)


def _tc_combine_stats(parts, hp, dis, b):
    def body(p0_ref, p1_ref, hp_ref, dis_ref, b_ref, conv_ref, stats_ref,
             acc_ref):
        i = pl.program_id(0)

        @pl.when(i == 0)
        def _():
            acc_ref[...] = jnp.zeros_like(acc_ref)

        conv = (p0_ref[...] + p1_ref[...] - hp_ref[...]) * dis_ref[...] \
            + b_ref[...]
        conv_ref[...] = conv
        rows = i * _BM + lax.broadcasted_iota(jnp.int32, (_BM, 1), 0)
        cm = jnp.where(rows < _N, conv, 0.0)
        acc_ref[0:1, :] += jnp.sum(cm, axis=0, keepdims=True)
        acc_ref[1:2, :] += jnp.sum(cm * conv, axis=0, keepdims=True)

        @pl.when(i == _G - 1)
        def _():
            stats_ref[...] = acc_ref[...]

    return pl.pallas_call(
        body,
        grid=(_G,),
        in_specs=[pl.BlockSpec((_BM, _D), lambda i: (i, 0)),
                  pl.BlockSpec((_BM, _D), lambda i: (i + _G, 0)),
                  pl.BlockSpec((_BM, _D), lambda i: (i, 0)),
                  pl.BlockSpec((_BM, 1), lambda i: (i, 0)),
                  pl.BlockSpec((1, _D), lambda i: (0, 0))],
        out_specs=[pl.BlockSpec((_BM, _D), lambda i: (i, 0)),
                   pl.BlockSpec((2, _D), lambda i: (0, 0))],
        out_shape=[jax.ShapeDtypeStruct((_NP, _D), jnp.float32),
                   jax.ShapeDtypeStruct((2, _D), jnp.float32)],
        scratch_shapes=[pltpu.VMEM((2, _D), jnp.float32)],
    )(parts, parts, hp, dis, b)


def _tc_bn_matmul(conv, stats, gamma, beta, w2, dis):
    def body(conv_ref, stats_ref, g_ref, be_ref, w_ref, dis_ref, o_ref):
        mean = stats_ref[0:1, :] * (1.0 / _N)
        var = stats_ref[1:2, :] * (1.0 / _N) - mean * mean
        istd = lax.rsqrt(var + _BN_EPS)
        y = (conv_ref[...] - mean) * (istd * g_ref[...]) + be_ref[...]
        y = jnp.maximum(y, 0.0)
        h2 = jnp.dot(y, w_ref[...], preferred_element_type=jnp.float32)
        o_ref[...] = h2 * dis_ref[...]

    return pl.pallas_call(
        body,
        grid=(_G,),
        in_specs=[pl.BlockSpec((_BM, _D), lambda i: (i, 0)),
                  pl.BlockSpec((2, _D), lambda i: (0, 0)),
                  pl.BlockSpec((1, _D), lambda i: (0, 0)),
                  pl.BlockSpec((1, _D), lambda i: (0, 0)),
                  pl.BlockSpec((_D, _D), lambda i: (0, 0)),
                  pl.BlockSpec((_BM, 1), lambda i: (i, 0))],
        out_specs=pl.BlockSpec((_BM, _D), lambda i: (i, 0)),
        out_shape=jax.ShapeDtypeStruct((_NP, _D), jnp.float32),
    )(conv, stats, gamma, beta, w2, dis)


def _tc_final(parts, hp, dis, b):
    def body(p0_ref, p1_ref, hp_ref, dis_ref, b_ref, o_ref):
        o_ref[...] = (p0_ref[...] + p1_ref[...] - hp_ref[...]) \
            * dis_ref[...] + b_ref[...]

    return pl.pallas_call(
        body,
        grid=(_G,),
        in_specs=[pl.BlockSpec((_BM, _D), lambda i: (i, 0)),
                  pl.BlockSpec((_BM, _D), lambda i: (i + _G, 0)),
                  pl.BlockSpec((_BM, _D), lambda i: (i, 0)),
                  pl.BlockSpec((_BM, 1), lambda i: (i, 0)),
                  pl.BlockSpec((1, _D), lambda i: (0, 0))],
        out_specs=pl.BlockSpec((_BM, _D), lambda i: (i, 0)),
        out_shape=jax.ShapeDtypeStruct((_NP, _D), jnp.float32),
    )(parts, parts, hp, dis, b)


def kernel(node_feat, edge_index, W1, b1, gamma, beta, W2, b2):
    src = edge_index[0]
    dst = edge_index[1]
    pad = jnp.full((_EP - _E,), _N, jnp.int32)
    src_rows = jnp.concatenate([src, pad]).reshape(_EROWS, _CHUNK)
    dst_rows = jnp.concatenate([dst, pad]).reshape(_EROWS, _CHUNK)
    x_pad = jnp.zeros((_NP, _D), jnp.float32).at[:_N].set(node_feat)
    b1r = b1.reshape(1, _D)
    b2r = b2.reshape(1, _D)
    gr = gamma.reshape(1, _D)
    ber = beta.reshape(1, _D)

    h1 = _tc_matmul(x_pad, W1)
    degp = _sc_degree(dst_rows)
    dis = _tc_deg_reduce(degp).reshape(_NP, 1)
    h1p = _tc_scale(h1, dis)
    parts1 = _sc_aggregate(h1p, src_rows, dst_rows)
    conv1, stats = _tc_combine_stats(parts1, h1p, dis, b1r)
    h2p = _tc_bn_matmul(conv1, stats, gr, ber, W2, dis)
    parts2 = _sc_aggregate(h2p, src_rows, dst_rows)
    out = _tc_final(parts2, h2p, dis, b2r)
    return out[:_N]


# speedup vs baseline: 10.9504x; 1.0000x over previous
import dataclasses
import functools

import jax
import jax.numpy as jnp
from jax import lax
from jax.experimental import pallas as pl
from jax.experimental.pallas import tpu as pltpu
from jax.experimental.pallas import tpu_sc as plsc

_N = 10000
_D = 128
_E = 320000
_NP = 10240
_CHUNK = 128
_NSUB = 16
_NCORE = 2
_NW = _NSUB * _NCORE
_EP = 327680
_EROWS = _EP // _CHUNK
_ROWS_W = _EROWS // _NW
_ACC_W = _NP // _NSUB
_BN_EPS = 1e-5
_BM = 1024
_G = _NP // _BM


def _sc_mesh():
    return plsc.VectorSubcoreMesh(core_axis_name="c", subcore_axis_name="s")


def _sc_compiler_params():
    cp = pltpu.CompilerParams()
    if "needs_layout_passes" in pltpu.CompilerParams.__dataclass_fields__:
        cp = dataclasses.replace(cp, needs_layout_passes=False)
    return cp


def _sc_degree(dst_rows):
    @functools.partial(
        pl.kernel,
        out_type=jax.ShapeDtypeStruct((_NW, _NP), jnp.float32),
        mesh=_sc_mesh(),
        compiler_params=_sc_compiler_params(),
        scratch_types=[
            pltpu.VMEM((_ROWS_W, _CHUNK), jnp.int32),
            pltpu.VMEM((_NP,), jnp.float32),
            pltpu.SemaphoreType.DMA,
            pltpu.SemaphoreType.DMA,
        ],
    )
    def k(dst_hbm, out_hbm, didx, hist, s0, s1):
        c = lax.axis_index("c")
        s = lax.axis_index("s")
        wid = c * _NSUB + s
        pltpu.async_copy(dst_hbm.at[pl.ds(wid * _ROWS_W, _ROWS_W)], didx,
                         s0).wait()

        @pl.loop(0, _NP // 16)
        def _(i):
            hist[pl.ds(i * 16, 16)] = jnp.zeros((16,), jnp.float32)

        @pl.loop(0, _ROWS_W)
        def _(j):
            @pl.loop(0, _CHUNK // 16)
            def _(kk):
                iv = didx[j, pl.ds(kk * 16, 16)]
                plsc.addupdate_scatter(hist, [iv],
                                       jnp.ones((16,), jnp.float32))

        pltpu.async_copy(hist, out_hbm.at[wid], s1).wait()

    return k(dst_rows)


def _tc_deg_reduce(deg_parts):
    def body(d_ref, o_ref):
        o_ref[...] = lax.rsqrt(
            jnp.sum(d_ref[...], axis=0, keepdims=True) + 1.0)

    return pl.pallas_call(
        body,
        grid=(_G,),
        in_specs=[pl.BlockSpec((_NW, _BM), lambda i: (0, i))],
        out_specs=pl.BlockSpec((1, _BM), lambda i: (0, i)),
        out_shape=jax.ShapeDtypeStruct((1, _NP), jnp.float32),
    )(deg_parts)


def _sc_aggregate(hp, src_rows, dst_rows):
    nbuf = 2
    nphase = 2
    phr = _ROWS_W // nphase

    @functools.partial(
        pl.kernel,
        out_type=jax.ShapeDtypeStruct((2 * _NP, _D), jnp.float32),
        mesh=_sc_mesh(),
        scratch_types=[
            pltpu.VMEM((phr, _CHUNK), jnp.int32),
            pltpu.VMEM((phr, _CHUNK), jnp.int32),
            pltpu.VMEM((_CHUNK, _D), jnp.float32),
            pltpu.VMEM((_CHUNK, _D), jnp.float32),
            pltpu.VMEM_SHARED((_NP, _D), jnp.float32),
            pltpu.SemaphoreType.DMA,
            pltpu.SemaphoreType.DMA,
            pltpu.SemaphoreType.DMA,
            pltpu.SemaphoreType.DMA,
        ],
    )
    def k(hp_hbm, src_hbm, dst_hbm, out_hbm, sidx, didx,
          buf0, buf1, acc, sm0, sm1, ss0, ss1):
        c = lax.axis_index("c")
        s = lax.axis_index("s")
        wid = c * _NSUB + s
        row0 = wid * _ROWS_W
        r0 = s * _ACC_W
        pltpu.sync_copy(hp_hbm.at[pl.ds(r0, _ACC_W)],
                        acc.at[pl.ds(r0, _ACC_W)])
        plsc.subcore_barrier()

        bufs = (buf0, buf1)
        gsem = (sm0, sm1)
        ssem = (ss0, ss1)
        for ph in range(nphase):
            pltpu.sync_copy(
                src_hbm.at[pl.ds(row0 + ph * phr, phr)], sidx)
            pltpu.sync_copy(
                dst_hbm.at[pl.ds(row0 + ph * phr, phr)], didx)
            for b in range(nbuf):
                pltpu.async_copy(hp_hbm.at[sidx.at[b]], bufs[b], gsem[b])

            @pl.loop(0, phr, step=nbuf)
            def _(j, ph=ph):
                for b in range(nbuf):
                    pltpu.make_async_copy(hp_hbm.at[sidx.at[j + b]],
                                          bufs[b], gsem[b]).wait()
                    pltpu.async_copy(bufs[b], acc.at[didx.at[j + b]],
                                     ssem[b], add=True)

                    @pl.when(j + b + nbuf < phr)
                    def _(b=b, j=j):
                        pltpu.make_async_copy(
                            bufs[b], acc.at[didx.at[j + b]],
                            ssem[b]).wait()
                        pltpu.async_copy(hp_hbm.at[sidx.at[j + b + nbuf]],
                                         bufs[b], gsem[b])

            for b in range(nbuf):
                last = phr - nbuf + b
                pltpu.make_async_copy(bufs[b], acc.at[didx.at[last]],
                                      ssem[b]).wait()

        plsc.subcore_barrier()
        pltpu.sync_copy(acc.at[pl.ds(r0, _ACC_W)],
                        out_hbm.at[pl.ds(c * _NP + r0, _ACC_W)])

    return k(hp, src_rows, dst_rows)


def _tc_matmul(x, w):
    def body(x_ref, w_ref, o_ref):
        o_ref[...] = jnp.dot(x_ref[...], w_ref[...],
                             preferred_element_type=jnp.float32)

    return pl.pallas_call(
        body,
        grid=(_G,),
        in_specs=[pl.BlockSpec((_BM, _D), lambda i: (i, 0)),
                  pl.BlockSpec((_D, _D), lambda i: (0, 0))],
        out_specs=pl.BlockSpec((_BM, _D), lambda i: (i, 0)),
        out_shape=jax.ShapeDtypeStruct((_NP, _D), jnp.float32),
    )(x, w)


def _tc_scale(h1, dis):
    def body(h_ref, dis_ref, hp_ref):
        hp_ref[...] = h_ref[...] * dis_ref[...]

    return pl.pallas_call(
        body,
        grid=(_G,),
        in_specs=[pl.BlockSpec((_BM, _D), lambda i: (i, 0)),
                  pl.BlockSpec((_BM, 1), lambda i: (i, 0))],
        out_specs=pl.BlockSpec((_BM, _D), lambda i: (i, 0)),
        out_shape=jax.ShapeDtypeStruct((_NP, _D), jnp.float32),
    )(h1, dis)


def _tc_combine_stats(parts, hp, dis, b):
    def body(p0_ref, p1_ref, hp_ref, dis_ref, b_ref, conv_ref, stats_ref,
             acc_ref):
        i = pl.program_id(0)

        @pl.when(i == 0)
        def _():
            acc_ref[...] = jnp.zeros_like(acc_ref)

        conv = (p0_ref[...] + p1_ref[...] - hp_ref[...]) * dis_ref[...] \
            + b_ref[...]
        conv_ref[...] = conv
        rows = i * _BM + lax.broadcasted_iota(jnp.int32, (_BM, 1), 0)
        cm = jnp.where(rows < _N, conv, 0.0)
        acc_ref[0:1, :] += jnp.sum(cm, axis=0, keepdims=True)
        acc_ref[1:2, :] += jnp.sum(cm * conv, axis=0, keepdims=True)

        @pl.when(i == _G - 1)
        def _():
            stats_ref[...] = acc_ref[...]

    return pl.pallas_call(
        body,
        grid=(_G,),
        in_specs=[pl.BlockSpec((_BM, _D), lambda i: (i, 0)),
                  pl.BlockSpec((_BM, _D), lambda i: (i + _G, 0)),
                  pl.BlockSpec((_BM, _D), lambda i: (i, 0)),
                  pl.BlockSpec((_BM, 1), lambda i: (i, 0)),
                  pl.BlockSpec((1, _D), lambda i: (0, 0))],
        out_specs=[pl.BlockSpec((_BM, _D), lambda i: (i, 0)),
                   pl.BlockSpec((2, _D), lambda i: (0, 0))],
        out_shape=[jax.ShapeDtypeStruct((_NP, _D), jnp.float32),
                   jax.ShapeDtypeStruct((2, _D), jnp.float32)],
        scratch_shapes=[pltpu.VMEM((2, _D), jnp.float32)],
    )(parts, parts, hp, dis, b)


def _tc_bn_matmul(conv, stats, gamma, beta, w2, dis):
    def body(conv_ref, stats_ref, g_ref, be_ref, w_ref, dis_ref, o_ref):
        mean = stats_ref[0:1, :] * (1.0 / _N)
        var = stats_ref[1:2, :] * (1.0 / _N) - mean * mean
        istd = lax.rsqrt(var + _BN_EPS)
        y = (conv_ref[...] - mean) * (istd * g_ref[...]) + be_ref[...]
        y = jnp.maximum(y, 0.0)
        h2 = jnp.dot(y, w_ref[...], preferred_element_type=jnp.float32)
        o_ref[...] = h2 * dis_ref[...]

    return pl.pallas_call(
        body,
        grid=(_G,),
        in_specs=[pl.BlockSpec((_BM, _D), lambda i: (i, 0)),
                  pl.BlockSpec((2, _D), lambda i: (0, 0)),
                  pl.BlockSpec((1, _D), lambda i: (0, 0)),
                  pl.BlockSpec((1, _D), lambda i: (0, 0)),
                  pl.BlockSpec((_D, _D), lambda i: (0, 0)),
                  pl.BlockSpec((_BM, 1), lambda i: (i, 0))],
        out_specs=pl.BlockSpec((_BM, _D), lambda i: (i, 0)),
        out_shape=jax.ShapeDtypeStruct((_NP, _D), jnp.float32),
    )(conv, stats, gamma, beta, w2, dis)


def _tc_final(parts, hp, dis, b):
    def body(p0_ref, p1_ref, hp_ref, dis_ref, b_ref, o_ref):
        o_ref[...] = (p0_ref[...] + p1_ref[...] - hp_ref[...]) \
            * dis_ref[...] + b_ref[...]

    return pl.pallas_call(
        body,
        grid=(_G,),
        in_specs=[pl.BlockSpec((_BM, _D), lambda i: (i, 0)),
                  pl.BlockSpec((_BM, _D), lambda i: (i + _G, 0)),
                  pl.BlockSpec((_BM, _D), lambda i: (i, 0)),
                  pl.BlockSpec((_BM, 1), lambda i: (i, 0)),
                  pl.BlockSpec((1, _D), lambda i: (0, 0))],
        out_specs=pl.BlockSpec((_BM, _D), lambda i: (i, 0)),
        out_shape=jax.ShapeDtypeStruct((_NP, _D), jnp.float32),
    )(parts, parts, hp, dis, b)


def kernel(node_feat, edge_index, W1, b1, gamma, beta, W2, b2):
    src = edge_index[0]
    dst = edge_index[1]
    pad = jnp.full((_EP - _E,), _N, jnp.int32)
    src_rows = jnp.concatenate([src, pad]).reshape(_EROWS, _CHUNK)
    dst_rows = jnp.concatenate([dst, pad]).reshape(_EROWS, _CHUNK)
    x_pad = jnp.zeros((_NP, _D), jnp.float32).at[:_N].set(node_feat)
    b1r = b1.reshape(1, _D)
    b2r = b2.reshape(1, _D)
    gr = gamma.reshape(1, _D)
    ber = beta.reshape(1, _D)

    h1 = _tc_matmul(x_pad, W1)
    degp = _sc_degree(dst_rows)
    dis = _tc_deg_reduce(degp).reshape(_NP, 1)
    h1p = _tc_scale(h1, dis)
    parts1 = _sc_aggregate(h1p, src_rows, dst_rows)
    conv1, stats = _tc_combine_stats(parts1, h1p, dis, b1r)
    h2p = _tc_bn_matmul(conv1, stats, gr, ber, W2, dis)
    parts2 = _sc_aggregate(h2p, src_rows, dst_rows)
    out = _tc_final(parts2, h2p, dis, b2r)
    return out[:_N]


# speedup vs baseline: 30.2541x; 2.7628x over previous
import dataclasses
import functools

import jax
import jax.numpy as jnp
from jax import lax
from jax.experimental import pallas as pl
from jax.experimental.pallas import tpu as pltpu
from jax.experimental.pallas import tpu_sc as plsc

_N = 10000
_D = 128
_E = 320000
_NP = 10240
_CHUNK = 128
_NSUB = 16
_NCORE = 2
_NW = _NSUB * _NCORE
_EP = 327680
_EROWS = _EP // _CHUNK
_ROWS_W = _EROWS // _NW
_ACC_W = _NP // _NSUB
_BN_EPS = 1e-5
_BM = 1024
_G = _NP // _BM


def _sc_mesh():
    return plsc.VectorSubcoreMesh(core_axis_name="c", subcore_axis_name="s")


def _sc_compiler_params():
    cp = pltpu.CompilerParams()
    if "needs_layout_passes" in pltpu.CompilerParams.__dataclass_fields__:
        cp = dataclasses.replace(cp, needs_layout_passes=False)
    return cp


def _sc_degree(dst_rows):
    @functools.partial(
        pl.kernel,
        out_type=jax.ShapeDtypeStruct((_NW, _NP), jnp.float32),
        mesh=_sc_mesh(),
        compiler_params=_sc_compiler_params(),
        scratch_types=[
            pltpu.VMEM((_ROWS_W, _CHUNK), jnp.int32),
            pltpu.VMEM((_NP,), jnp.float32),
            pltpu.SemaphoreType.DMA,
            pltpu.SemaphoreType.DMA,
        ],
    )
    def k(dst_hbm, out_hbm, didx, hist, s0, s1):
        c = lax.axis_index("c")
        s = lax.axis_index("s")
        wid = c * _NSUB + s
        pltpu.async_copy(dst_hbm.at[pl.ds(wid * _ROWS_W, _ROWS_W)], didx,
                         s0).wait()

        @pl.loop(0, _NP // 16)
        def _(i):
            hist[pl.ds(i * 16, 16)] = jnp.zeros((16,), jnp.float32)

        @pl.loop(0, _ROWS_W)
        def _(j):
            @pl.loop(0, _CHUNK // 16)
            def _(kk):
                iv = didx[j, pl.ds(kk * 16, 16)]
                plsc.addupdate_scatter(hist, [iv],
                                       jnp.ones((16,), jnp.float32))

        pltpu.async_copy(hist, out_hbm.at[wid], s1).wait()

    return k(dst_rows)


def _tc_deg_reduce(deg_parts):
    def body(d_ref, o_ref):
        o_ref[...] = lax.rsqrt(
            jnp.sum(d_ref[...], axis=0, keepdims=True) + 1.0)

    return pl.pallas_call(
        body,
        grid=(_G,),
        in_specs=[pl.BlockSpec((_NW, _BM), lambda i: (0, i))],
        out_specs=pl.BlockSpec((1, _BM), lambda i: (0, i)),
        out_shape=jax.ShapeDtypeStruct((1, _NP), jnp.float32),
    )(deg_parts)


def _sc_aggregate(hp, src_rows, dst_rows):
    nbuf = 2
    nphase = 2
    phr = _ROWS_W // nphase

    @functools.partial(
        pl.kernel,
        out_type=jax.ShapeDtypeStruct((2 * _NP, _D), jnp.float32),
        mesh=_sc_mesh(),
        scratch_types=[
            pltpu.VMEM((phr, _CHUNK), jnp.int32),
            pltpu.VMEM((phr, _CHUNK), jnp.int32),
            pltpu.VMEM((_CHUNK, _D), jnp.float32),
            pltpu.VMEM((_CHUNK, _D), jnp.float32),
            pltpu.VMEM_SHARED((_NP, _D), jnp.float32),
            pltpu.SemaphoreType.DMA,
            pltpu.SemaphoreType.DMA,
            pltpu.SemaphoreType.DMA,
            pltpu.SemaphoreType.DMA,
        ],
    )
    def k(hp_hbm, src_hbm, dst_hbm, out_hbm, sidx, didx,
          buf0, buf1, acc, sm0, sm1, ss0, ss1):
        c = lax.axis_index("c")
        s = lax.axis_index("s")
        wid = c * _NSUB + s
        row0 = wid * _ROWS_W
        r0 = s * _ACC_W
        pltpu.sync_copy(hp_hbm.at[pl.ds(r0, _ACC_W)],
                        acc.at[pl.ds(r0, _ACC_W)])
        plsc.subcore_barrier()

        bufs = (buf0, buf1)
        gsem = (sm0, sm1)
        ssem = (ss0, ss1)
        for ph in range(nphase):
            pltpu.sync_copy(
                src_hbm.at[pl.ds(row0 + ph * phr, phr)], sidx)
            pltpu.sync_copy(
                dst_hbm.at[pl.ds(row0 + ph * phr, phr)], didx)
            for b in range(nbuf):
                pltpu.async_copy(hp_hbm.at[sidx.at[b]], bufs[b], gsem[b])

            @pl.loop(0, phr, step=nbuf)
            def _(j, ph=ph):
                for b in range(nbuf):
                    pltpu.make_async_copy(hp_hbm.at[sidx.at[j + b]],
                                          bufs[b], gsem[b]).wait()
                    pltpu.async_copy(bufs[b], acc.at[didx.at[j + b]],
                                     ssem[b], add=True)

                    @pl.when(j + b + nbuf < phr)
                    def _(b=b, j=j):
                        pltpu.make_async_copy(
                            bufs[b], acc.at[didx.at[j + b]],
                            ssem[b]).wait()
                        pltpu.async_copy(hp_hbm.at[sidx.at[j + b + nbuf]],
                                         bufs[b], gsem[b])

            for b in range(nbuf):
                last = phr - nbuf + b
                pltpu.make_async_copy(bufs[b], acc.at[didx.at[last]],
                                      ssem[b]).wait()

        plsc.subcore_barrier()
        pltpu.sync_copy(acc.at[pl.ds(r0, _ACC_W)],
                        out_hbm.at[pl.ds(c * _NP + r0, _ACC_W)])

    return k(hp, src_rows, dst_rows)


def _tc_matmul(x, w):
    def body(x_ref, w_ref, o_ref):
        o_ref[...] = jnp.dot(x_ref[...], w_ref[...],
                             preferred_element_type=jnp.float32)

    return pl.pallas_call(
        body,
        grid=(_G,),
        in_specs=[pl.BlockSpec((_BM, _D), lambda i: (i, 0)),
                  pl.BlockSpec((_D, _D), lambda i: (0, 0))],
        out_specs=pl.BlockSpec((_BM, _D), lambda i: (i, 0)),
        out_shape=jax.ShapeDtypeStruct((_NP, _D), jnp.float32),
    )(x, w)


def _tc_scale(h1, dis):
    def body(h_ref, dis_ref, hp_ref):
        hp_ref[...] = h_ref[...] * dis_ref[...]

    return pl.pallas_call(
        body,
        grid=(_G,),
        in_specs=[pl.BlockSpec((_BM, _D), lambda i: (i, 0)),
                  pl.BlockSpec((_BM, 1), lambda i: (i, 0))],
        out_specs=pl.BlockSpec((_BM, _D), lambda i: (i, 0)),
        out_shape=jax.ShapeDtypeStruct((_NP, _D), jnp.float32),
    )(h1, dis)


def _tc_combine_stats(parts, hp, dis, b):
    def body(p0_ref, p1_ref, hp_ref, dis_ref, b_ref, conv_ref, stats_ref,
             acc_ref):
        i = pl.program_id(0)

        @pl.when(i == 0)
        def _():
            acc_ref[...] = jnp.zeros_like(acc_ref)

        conv = (p0_ref[...] + p1_ref[...] - hp_ref[...]) * dis_ref[...] \
            + b_ref[...]
        conv_ref[...] = conv
        rows = i * _BM + lax.broadcasted_iota(jnp.int32, (_BM, 1), 0)
        cm = jnp.where(rows < _N, conv, 0.0)
        acc_ref[0:1, :] += jnp.sum(cm, axis=0, keepdims=True)
        acc_ref[1:2, :] += jnp.sum(cm * conv, axis=0, keepdims=True)

        @pl.when(i == _G - 1)
        def _():
            stats_ref[...] = acc_ref[...]

    return pl.pallas_call(
        body,
        grid=(_G,),
        in_specs=[pl.BlockSpec((_BM, _D), lambda i: (i, 0)),
                  pl.BlockSpec((_BM, _D), lambda i: (i + _G, 0)),
                  pl.BlockSpec((_BM, _D), lambda i: (i, 0)),
                  pl.BlockSpec((_BM, 1), lambda i: (i, 0)),
                  pl.BlockSpec((1, _D), lambda i: (0, 0))],
        out_specs=[pl.BlockSpec((_BM, _D), lambda i: (i, 0)),
                   pl.BlockSpec((2, _D), lambda i: (0, 0))],
        out_shape=[jax.ShapeDtypeStruct((_NP, _D), jnp.float32),
                   jax.ShapeDtypeStruct((2, _D), jnp.float32)],
        scratch_shapes=[pltpu.VMEM((2, _D), jnp.float32)],
    )(parts, parts, hp, dis, b)


def _tc_bn_matmul(conv, stats, gamma, beta, w2, dis):
    def body(conv_ref, stats_ref, g_ref, be_ref, w_ref, dis_ref, o_ref):
        mean = stats_ref[0:1, :] * (1.0 / _N)
        var = stats_ref[1:2, :] * (1.0 / _N) - mean * mean
        istd = lax.rsqrt(var + _BN_EPS)
        y = (conv_ref[...] - mean) * (istd * g_ref[...]) + be_ref[...]
        y = jnp.maximum(y, 0.0)
        h2 = jnp.dot(y, w_ref[...], preferred_element_type=jnp.float32)
        o_ref[...] = h2 * dis_ref[...]

    return pl.pallas_call(
        body,
        grid=(_G,),
        in_specs=[pl.BlockSpec((_BM, _D), lambda i: (i, 0)),
                  pl.BlockSpec((2, _D), lambda i: (0, 0)),
                  pl.BlockSpec((1, _D), lambda i: (0, 0)),
                  pl.BlockSpec((1, _D), lambda i: (0, 0)),
                  pl.BlockSpec((_D, _D), lambda i: (0, 0)),
                  pl.BlockSpec((_BM, 1), lambda i: (i, 0))],
        out_specs=pl.BlockSpec((_BM, _D), lambda i: (i, 0)),
        out_shape=jax.ShapeDtypeStruct((_NP, _D), jnp.float32),
    )(conv, stats, gamma, beta, w2, dis)


def _tc_final(parts, hp, dis, b):
    def body(p0_ref, p1_ref, hp_ref, dis_ref, b_ref, o_ref):
        o_ref[...] = (p0_ref[...] + p1_ref[...] - hp_ref[...]) \
            * dis_ref[...] + b_ref[...]

    return pl.pallas_call(
        body,
        grid=(_G,),
        in_specs=[pl.BlockSpec((_BM, _D), lambda i: (i, 0)),
                  pl.BlockSpec((_BM, _D), lambda i: (i + _G, 0)),
                  pl.BlockSpec((_BM, _D), lambda i: (i, 0)),
                  pl.BlockSpec((_BM, 1), lambda i: (i, 0)),
                  pl.BlockSpec((1, _D), lambda i: (0, 0))],
        out_specs=pl.BlockSpec((_BM, _D), lambda i: (i, 0)),
        out_shape=jax.ShapeDtypeStruct((_NP, _D), jnp.float32),
    )(parts, parts, hp, dis, b)


def kernel(node_feat, edge_index, W1, b1, gamma, beta, W2, b2):
    src = edge_index[0]
    dst = edge_index[1]
    pad = _N + (jnp.arange(_EP - _E, dtype=jnp.int32) % (_NP - _N))
    src_rows = jnp.concatenate([src, pad]).reshape(_EROWS, _CHUNK)
    dst_rows = jnp.concatenate([dst, pad]).reshape(_EROWS, _CHUNK)
    x_pad = jnp.zeros((_NP, _D), jnp.float32).at[:_N].set(node_feat)
    b1r = b1.reshape(1, _D)
    b2r = b2.reshape(1, _D)
    gr = gamma.reshape(1, _D)
    ber = beta.reshape(1, _D)

    h1 = _tc_matmul(x_pad, W1)
    degp = _sc_degree(dst_rows)
    dis = _tc_deg_reduce(degp).reshape(_NP, 1)
    h1p = _tc_scale(h1, dis)
    parts1 = _sc_aggregate(h1p, src_rows, dst_rows)
    conv1, stats = _tc_combine_stats(parts1, h1p, dis, b1r)
    h2p = _tc_bn_matmul(conv1, stats, gr, ber, W2, dis)
    parts2 = _sc_aggregate(h2p, src_rows, dst_rows)
    out = _tc_final(parts2, h2p, dis, b2r)
    return out[:_N]


# speedup vs baseline: 31.3993x; 1.0379x over previous
import dataclasses
import functools

import jax
import jax.numpy as jnp
from jax import lax
from jax.experimental import pallas as pl
from jax.experimental.pallas import tpu as pltpu
from jax.experimental.pallas import tpu_sc as plsc

_N = 10000
_D = 128
_E = 320000
_NP = 10240
_CHUNK = 64
_NSUB = 16
_NCORE = 2
_NW = _NSUB * _NCORE
_EP = 327680
_EROWS = _EP // _CHUNK
_ROWS_W = _EROWS // _NW
_NBUF = 4
_NPHASE = 4
_ACC_W = _NP // _NSUB
_BN_EPS = 1e-5
_BM = 1024
_G = _NP // _BM


def _sc_mesh():
    return plsc.VectorSubcoreMesh(core_axis_name="c", subcore_axis_name="s")


def _sc_compiler_params():
    cp = pltpu.CompilerParams()
    if "needs_layout_passes" in pltpu.CompilerParams.__dataclass_fields__:
        cp = dataclasses.replace(cp, needs_layout_passes=False)
    return cp


def _sc_degree(dst_rows):
    @functools.partial(
        pl.kernel,
        out_type=jax.ShapeDtypeStruct((_NW, _NP), jnp.float32),
        mesh=_sc_mesh(),
        compiler_params=_sc_compiler_params(),
        scratch_types=[
            pltpu.VMEM((_ROWS_W, _CHUNK), jnp.int32),
            pltpu.VMEM((_NP,), jnp.float32),
            pltpu.SemaphoreType.DMA,
            pltpu.SemaphoreType.DMA,
        ],
    )
    def k(dst_hbm, out_hbm, didx, hist, s0, s1):
        c = lax.axis_index("c")
        s = lax.axis_index("s")
        wid = c * _NSUB + s
        pltpu.async_copy(dst_hbm.at[pl.ds(wid * _ROWS_W, _ROWS_W)], didx,
                         s0).wait()

        @pl.loop(0, _NP // 16)
        def _(i):
            hist[pl.ds(i * 16, 16)] = jnp.zeros((16,), jnp.float32)

        @pl.loop(0, _ROWS_W)
        def _(j):
            @pl.loop(0, _CHUNK // 16)
            def _(kk):
                iv = didx[j, pl.ds(kk * 16, 16)]
                plsc.addupdate_scatter(hist, [iv],
                                       jnp.ones((16,), jnp.float32))

        pltpu.async_copy(hist, out_hbm.at[wid], s1).wait()

    return k(dst_rows)


def _tc_deg_reduce(deg_parts):
    def body(d_ref, o_ref):
        o_ref[...] = lax.rsqrt(
            jnp.sum(d_ref[...], axis=0, keepdims=True) + 1.0)

    return pl.pallas_call(
        body,
        grid=(_G,),
        in_specs=[pl.BlockSpec((_NW, _BM), lambda i: (0, i))],
        out_specs=pl.BlockSpec((1, _BM), lambda i: (0, i)),
        out_shape=jax.ShapeDtypeStruct((1, _NP), jnp.float32),
    )(deg_parts)


def _sc_aggregate(hp, src_rows, dst_rows):
    nbuf = _NBUF
    nphase = _NPHASE
    phr = _ROWS_W // nphase

    @functools.partial(
        pl.kernel,
        out_type=jax.ShapeDtypeStruct((2 * _NP, _D), jnp.float32),
        mesh=_sc_mesh(),
        scratch_types=(
            [pltpu.VMEM((phr, _CHUNK), jnp.int32),
             pltpu.VMEM((phr, _CHUNK), jnp.int32)]
            + [pltpu.VMEM((_CHUNK, _D), jnp.float32)] * nbuf
            + [pltpu.VMEM_SHARED((_NP, _D), jnp.float32)]
            + [pltpu.SemaphoreType.DMA] * (2 * nbuf)
        ),
    )
    def k(hp_hbm, src_hbm, dst_hbm, out_hbm, sidx, didx, *rest):
        bufs = rest[:nbuf]
        acc = rest[nbuf]
        gsem = rest[nbuf + 1:2 * nbuf + 1]
        ssem = rest[2 * nbuf + 1:]
        c = lax.axis_index("c")
        s = lax.axis_index("s")
        wid = c * _NSUB + s
        row0 = wid * _ROWS_W
        r0 = s * _ACC_W
        pltpu.sync_copy(hp_hbm.at[pl.ds(r0, _ACC_W)],
                        acc.at[pl.ds(r0, _ACC_W)])
        plsc.subcore_barrier()

        for ph in range(nphase):
            pltpu.sync_copy(
                src_hbm.at[pl.ds(row0 + ph * phr, phr)], sidx)
            pltpu.sync_copy(
                dst_hbm.at[pl.ds(row0 + ph * phr, phr)], didx)
            for b in range(nbuf):
                pltpu.async_copy(hp_hbm.at[sidx.at[b]], bufs[b], gsem[b])

            @pl.loop(0, phr, step=nbuf)
            def _(j, ph=ph):
                for b in range(nbuf):
                    pltpu.make_async_copy(hp_hbm.at[sidx.at[j + b]],
                                          bufs[b], gsem[b]).wait()
                    pltpu.async_copy(bufs[b], acc.at[didx.at[j + b]],
                                     ssem[b], add=True)

                    @pl.when(j + b + nbuf < phr)
                    def _(b=b, j=j):
                        pltpu.make_async_copy(
                            bufs[b], acc.at[didx.at[j + b]],
                            ssem[b]).wait()
                        pltpu.async_copy(hp_hbm.at[sidx.at[j + b + nbuf]],
                                         bufs[b], gsem[b])

            for b in range(nbuf):
                last = phr - nbuf + b
                pltpu.make_async_copy(bufs[b], acc.at[didx.at[last]],
                                      ssem[b]).wait()

        plsc.subcore_barrier()
        pltpu.sync_copy(acc.at[pl.ds(r0, _ACC_W)],
                        out_hbm.at[pl.ds(c * _NP + r0, _ACC_W)])

    return k(hp, src_rows, dst_rows)


def _tc_matmul(x, w):
    def body(x_ref, w_ref, o_ref):
        o_ref[...] = jnp.dot(x_ref[...], w_ref[...],
                             preferred_element_type=jnp.float32)

    return pl.pallas_call(
        body,
        grid=(_G,),
        in_specs=[pl.BlockSpec((_BM, _D), lambda i: (i, 0)),
                  pl.BlockSpec((_D, _D), lambda i: (0, 0))],
        out_specs=pl.BlockSpec((_BM, _D), lambda i: (i, 0)),
        out_shape=jax.ShapeDtypeStruct((_NP, _D), jnp.float32),
    )(x, w)


def _tc_scale(h1, dis):
    def body(h_ref, dis_ref, hp_ref):
        hp_ref[...] = h_ref[...] * dis_ref[...]

    return pl.pallas_call(
        body,
        grid=(_G,),
        in_specs=[pl.BlockSpec((_BM, _D), lambda i: (i, 0)),
                  pl.BlockSpec((_BM, 1), lambda i: (i, 0))],
        out_specs=pl.BlockSpec((_BM, _D), lambda i: (i, 0)),
        out_shape=jax.ShapeDtypeStruct((_NP, _D), jnp.float32),
    )(h1, dis)


def _tc_combine_stats(parts, hp, dis, b):
    def body(p0_ref, p1_ref, hp_ref, dis_ref, b_ref, conv_ref, stats_ref,
             acc_ref):
        i = pl.program_id(0)

        @pl.when(i == 0)
        def _():
            acc_ref[...] = jnp.zeros_like(acc_ref)

        conv = (p0_ref[...] + p1_ref[...] - hp_ref[...]) * dis_ref[...] \
            + b_ref[...]
        conv_ref[...] = conv
        rows = i * _BM + lax.broadcasted_iota(jnp.int32, (_BM, 1), 0)
        cm = jnp.where(rows < _N, conv, 0.0)
        acc_ref[0:1, :] += jnp.sum(cm, axis=0, keepdims=True)
        acc_ref[1:2, :] += jnp.sum(cm * conv, axis=0, keepdims=True)

        @pl.when(i == _G - 1)
        def _():
            stats_ref[...] = acc_ref[...]

    return pl.pallas_call(
        body,
        grid=(_G,),
        in_specs=[pl.BlockSpec((_BM, _D), lambda i: (i, 0)),
                  pl.BlockSpec((_BM, _D), lambda i: (i + _G, 0)),
                  pl.BlockSpec((_BM, _D), lambda i: (i, 0)),
                  pl.BlockSpec((_BM, 1), lambda i: (i, 0)),
                  pl.BlockSpec((1, _D), lambda i: (0, 0))],
        out_specs=[pl.BlockSpec((_BM, _D), lambda i: (i, 0)),
                   pl.BlockSpec((2, _D), lambda i: (0, 0))],
        out_shape=[jax.ShapeDtypeStruct((_NP, _D), jnp.float32),
                   jax.ShapeDtypeStruct((2, _D), jnp.float32)],
        scratch_shapes=[pltpu.VMEM((2, _D), jnp.float32)],
    )(parts, parts, hp, dis, b)


def _tc_bn_matmul(conv, stats, gamma, beta, w2, dis):
    def body(conv_ref, stats_ref, g_ref, be_ref, w_ref, dis_ref, o_ref):
        mean = stats_ref[0:1, :] * (1.0 / _N)
        var = stats_ref[1:2, :] * (1.0 / _N) - mean * mean
        istd = lax.rsqrt(var + _BN_EPS)
        y = (conv_ref[...] - mean) * (istd * g_ref[...]) + be_ref[...]
        y = jnp.maximum(y, 0.0)
        h2 = jnp.dot(y, w_ref[...], preferred_element_type=jnp.float32)
        o_ref[...] = h2 * dis_ref[...]

    return pl.pallas_call(
        body,
        grid=(_G,),
        in_specs=[pl.BlockSpec((_BM, _D), lambda i: (i, 0)),
                  pl.BlockSpec((2, _D), lambda i: (0, 0)),
                  pl.BlockSpec((1, _D), lambda i: (0, 0)),
                  pl.BlockSpec((1, _D), lambda i: (0, 0)),
                  pl.BlockSpec((_D, _D), lambda i: (0, 0)),
                  pl.BlockSpec((_BM, 1), lambda i: (i, 0))],
        out_specs=pl.BlockSpec((_BM, _D), lambda i: (i, 0)),
        out_shape=jax.ShapeDtypeStruct((_NP, _D), jnp.float32),
    )(conv, stats, gamma, beta, w2, dis)


def _tc_final(parts, hp, dis, b):
    def body(p0_ref, p1_ref, hp_ref, dis_ref, b_ref, o_ref):
        o_ref[...] = (p0_ref[...] + p1_ref[...] - hp_ref[...]) \
            * dis_ref[...] + b_ref[...]

    return pl.pallas_call(
        body,
        grid=(_G,),
        in_specs=[pl.BlockSpec((_BM, _D), lambda i: (i, 0)),
                  pl.BlockSpec((_BM, _D), lambda i: (i + _G, 0)),
                  pl.BlockSpec((_BM, _D), lambda i: (i, 0)),
                  pl.BlockSpec((_BM, 1), lambda i: (i, 0)),
                  pl.BlockSpec((1, _D), lambda i: (0, 0))],
        out_specs=pl.BlockSpec((_BM, _D), lambda i: (i, 0)),
        out_shape=jax.ShapeDtypeStruct((_NP, _D), jnp.float32),
    )(parts, parts, hp, dis, b)


def kernel(node_feat, edge_index, W1, b1, gamma, beta, W2, b2):
    src = edge_index[0]
    dst = edge_index[1]
    pad = _N + (jnp.arange(_EP - _E, dtype=jnp.int32) % (_NP - _N))
    src_rows = jnp.concatenate([src, pad]).reshape(_EROWS, _CHUNK)
    dst_rows = jnp.concatenate([dst, pad]).reshape(_EROWS, _CHUNK)
    x_pad = jnp.zeros((_NP, _D), jnp.float32).at[:_N].set(node_feat)
    b1r = b1.reshape(1, _D)
    b2r = b2.reshape(1, _D)
    gr = gamma.reshape(1, _D)
    ber = beta.reshape(1, _D)

    h1 = _tc_matmul(x_pad, W1)
    degp = _sc_degree(dst_rows)
    dis = _tc_deg_reduce(degp).reshape(_NP, 1)
    h1p = _tc_scale(h1, dis)
    parts1 = _sc_aggregate(h1p, src_rows, dst_rows)
    conv1, stats = _tc_combine_stats(parts1, h1p, dis, b1r)
    h2p = _tc_bn_matmul(conv1, stats, gr, ber, W2, dis)
    parts2 = _sc_aggregate(h2p, src_rows, dst_rows)
    out = _tc_final(parts2, h2p, dis, b2r)
    return out[:_N]


# speedup vs baseline: 32.3084x; 1.0290x over previous
import dataclasses
import functools

import jax
import jax.numpy as jnp
from jax import lax
from jax.experimental import pallas as pl
from jax.experimental.pallas import tpu as pltpu
from jax.experimental.pallas import tpu_sc as plsc

_N = 10000
_D = 128
_E = 320000
_NP = 10240
_CHUNK = 64
_NSUB = 16
_NCORE = 2
_NW = _NSUB * _NCORE
_EP = 327680
_EROWS = _EP // _CHUNK
_ROWS_W = _EROWS // _NW
_NBUF = 4
_NPHASE = 4
_ACC_W = _NP // _NSUB
_BN_EPS = 1e-5
_BM = 1024
_G = _NP // _BM


def _sc_mesh():
    return plsc.VectorSubcoreMesh(core_axis_name="c", subcore_axis_name="s")


def _sc_compiler_params():
    cp = pltpu.CompilerParams()
    if "needs_layout_passes" in pltpu.CompilerParams.__dataclass_fields__:
        cp = dataclasses.replace(cp, needs_layout_passes=False)
    return cp


def _sc_degree(dst_rows):
    @functools.partial(
        pl.kernel,
        out_type=jax.ShapeDtypeStruct((_NW, _NP), jnp.float32),
        mesh=_sc_mesh(),
        compiler_params=_sc_compiler_params(),
        scratch_types=[
            pltpu.VMEM((_ROWS_W, _CHUNK), jnp.int32),
            pltpu.VMEM((_NP,), jnp.float32),
            pltpu.SemaphoreType.DMA,
            pltpu.SemaphoreType.DMA,
        ],
    )
    def k(dst_hbm, out_hbm, didx, hist, s0, s1):
        c = lax.axis_index("c")
        s = lax.axis_index("s")
        wid = c * _NSUB + s
        pltpu.async_copy(dst_hbm.at[pl.ds(wid * _ROWS_W, _ROWS_W)], didx,
                         s0).wait()

        @pl.loop(0, _NP // 16)
        def _(i):
            hist[pl.ds(i * 16, 16)] = jnp.zeros((16,), jnp.float32)

        @pl.loop(0, _ROWS_W)
        def _(j):
            @pl.loop(0, _CHUNK // 16)
            def _(kk):
                iv = didx[j, pl.ds(kk * 16, 16)]
                plsc.addupdate_scatter(hist, [iv],
                                       jnp.ones((16,), jnp.float32))

        pltpu.async_copy(hist, out_hbm.at[wid], s1).wait()

    return k(dst_rows)


def _tc_deg_reduce(deg_parts):
    def body(d_ref, o_ref):
        o_ref[...] = lax.rsqrt(
            jnp.sum(d_ref[...], axis=0, keepdims=True) + 1.0)

    return pl.pallas_call(
        body,
        grid=(_G,),
        in_specs=[pl.BlockSpec((_NW, _BM), lambda i: (0, i))],
        out_specs=pl.BlockSpec((1, _BM), lambda i: (0, i)),
        out_shape=jax.ShapeDtypeStruct((1, _NP), jnp.float32),
    )(deg_parts)


def _sc_aggregate(hp, src_rows, dst_rows):
    nbuf = _NBUF
    nphase = _NPHASE
    phr = _ROWS_W // nphase

    @functools.partial(
        pl.kernel,
        out_type=jax.ShapeDtypeStruct((2 * _NP, _D), jnp.float32),
        mesh=_sc_mesh(),
        scratch_types=(
            [pltpu.VMEM((phr, _CHUNK), jnp.int32),
             pltpu.VMEM((phr, _CHUNK), jnp.int32)]
            + [pltpu.VMEM((_CHUNK, _D), jnp.float32)] * nbuf
            + [pltpu.VMEM_SHARED((_NP, _D), jnp.float32)]
            + [pltpu.SemaphoreType.DMA] * (2 * nbuf)
        ),
    )
    def k(hp_hbm, src_hbm, dst_hbm, out_hbm, sidx, didx, *rest):
        bufs = rest[:nbuf]
        acc = rest[nbuf]
        gsem = rest[nbuf + 1:2 * nbuf + 1]
        ssem = rest[2 * nbuf + 1:]
        c = lax.axis_index("c")
        s = lax.axis_index("s")
        wid = c * _NSUB + s
        row0 = wid * _ROWS_W
        r0 = s * _ACC_W

        for ph in range(nphase):
            pltpu.sync_copy(
                src_hbm.at[pl.ds(row0 + ph * phr, phr)], sidx)
            pltpu.sync_copy(
                dst_hbm.at[pl.ds(row0 + ph * phr, phr)], didx)
            for b in range(nbuf):
                pltpu.async_copy(hp_hbm.at[sidx.at[b]], bufs[b], gsem[b])
            if ph == 0:
                pltpu.sync_copy(hp_hbm.at[pl.ds(r0, _ACC_W)],
                                acc.at[pl.ds(r0, _ACC_W)])
                plsc.subcore_barrier()

            @pl.loop(0, phr, step=nbuf)
            def _(j, ph=ph):
                for b in range(nbuf):
                    pltpu.make_async_copy(hp_hbm.at[sidx.at[j + b]],
                                          bufs[b], gsem[b]).wait()
                    pltpu.async_copy(bufs[b], acc.at[didx.at[j + b]],
                                     ssem[b], add=True)

                    @pl.when(j + b + nbuf < phr)
                    def _(b=b, j=j):
                        pltpu.make_async_copy(
                            bufs[b], acc.at[didx.at[j + b]],
                            ssem[b]).wait()
                        pltpu.async_copy(hp_hbm.at[sidx.at[j + b + nbuf]],
                                         bufs[b], gsem[b])

            for b in range(nbuf):
                last = phr - nbuf + b
                pltpu.make_async_copy(bufs[b], acc.at[didx.at[last]],
                                      ssem[b]).wait()

        plsc.subcore_barrier()
        pltpu.sync_copy(acc.at[pl.ds(r0, _ACC_W)],
                        out_hbm.at[pl.ds(c * _NP + r0, _ACC_W)])

    return k(hp, src_rows, dst_rows)


def _tc_matmul_scale(x, w, dis):
    def body(x_ref, w_ref, dis_ref, o_ref):
        o_ref[...] = jnp.dot(x_ref[...], w_ref[...],
                             preferred_element_type=jnp.float32) \
            * dis_ref[...]

    return pl.pallas_call(
        body,
        grid=(_G,),
        in_specs=[pl.BlockSpec((_BM, _D), lambda i: (i, 0)),
                  pl.BlockSpec((_D, _D), lambda i: (0, 0)),
                  pl.BlockSpec((_BM, 1), lambda i: (i, 0))],
        out_specs=pl.BlockSpec((_BM, _D), lambda i: (i, 0)),
        out_shape=jax.ShapeDtypeStruct((_NP, _D), jnp.float32),
    )(x, w, dis)


def _tc_combine_stats(parts, hp, dis, b):
    def body(p0_ref, p1_ref, hp_ref, dis_ref, b_ref, conv_ref, stats_ref,
             acc_ref):
        i = pl.program_id(0)

        @pl.when(i == 0)
        def _():
            acc_ref[...] = jnp.zeros_like(acc_ref)

        conv = (p0_ref[...] + p1_ref[...] - hp_ref[...]) * dis_ref[...] \
            + b_ref[...]
        conv_ref[...] = conv
        rows = i * _BM + lax.broadcasted_iota(jnp.int32, (_BM, 1), 0)
        cm = jnp.where(rows < _N, conv, 0.0)
        acc_ref[0:1, :] += jnp.sum(cm, axis=0, keepdims=True)
        acc_ref[1:2, :] += jnp.sum(cm * conv, axis=0, keepdims=True)

        @pl.when(i == _G - 1)
        def _():
            stats_ref[...] = acc_ref[...]

    return pl.pallas_call(
        body,
        grid=(_G,),
        in_specs=[pl.BlockSpec((_BM, _D), lambda i: (i, 0)),
                  pl.BlockSpec((_BM, _D), lambda i: (i + _G, 0)),
                  pl.BlockSpec((_BM, _D), lambda i: (i, 0)),
                  pl.BlockSpec((_BM, 1), lambda i: (i, 0)),
                  pl.BlockSpec((1, _D), lambda i: (0, 0))],
        out_specs=[pl.BlockSpec((_BM, _D), lambda i: (i, 0)),
                   pl.BlockSpec((2, _D), lambda i: (0, 0))],
        out_shape=[jax.ShapeDtypeStruct((_NP, _D), jnp.float32),
                   jax.ShapeDtypeStruct((2, _D), jnp.float32)],
        scratch_shapes=[pltpu.VMEM((2, _D), jnp.float32)],
    )(parts, parts, hp, dis, b)


def _tc_bn_matmul(conv, stats, gamma, beta, w2, dis):
    def body(conv_ref, stats_ref, g_ref, be_ref, w_ref, dis_ref, o_ref):
        mean = stats_ref[0:1, :] * (1.0 / _N)
        var = stats_ref[1:2, :] * (1.0 / _N) - mean * mean
        istd = lax.rsqrt(var + _BN_EPS)
        y = (conv_ref[...] - mean) * (istd * g_ref[...]) + be_ref[...]
        y = jnp.maximum(y, 0.0)
        h2 = jnp.dot(y, w_ref[...], preferred_element_type=jnp.float32)
        o_ref[...] = h2 * dis_ref[...]

    return pl.pallas_call(
        body,
        grid=(_G,),
        in_specs=[pl.BlockSpec((_BM, _D), lambda i: (i, 0)),
                  pl.BlockSpec((2, _D), lambda i: (0, 0)),
                  pl.BlockSpec((1, _D), lambda i: (0, 0)),
                  pl.BlockSpec((1, _D), lambda i: (0, 0)),
                  pl.BlockSpec((_D, _D), lambda i: (0, 0)),
                  pl.BlockSpec((_BM, 1), lambda i: (i, 0))],
        out_specs=pl.BlockSpec((_BM, _D), lambda i: (i, 0)),
        out_shape=jax.ShapeDtypeStruct((_NP, _D), jnp.float32),
    )(conv, stats, gamma, beta, w2, dis)


def _tc_final(parts, hp, dis, b):
    def body(p0_ref, p1_ref, hp_ref, dis_ref, b_ref, o_ref):
        o_ref[...] = (p0_ref[...] + p1_ref[...] - hp_ref[...]) \
            * dis_ref[...] + b_ref[...]

    return pl.pallas_call(
        body,
        grid=(_G,),
        in_specs=[pl.BlockSpec((_BM, _D), lambda i: (i, 0)),
                  pl.BlockSpec((_BM, _D), lambda i: (i + _G, 0)),
                  pl.BlockSpec((_BM, _D), lambda i: (i, 0)),
                  pl.BlockSpec((_BM, 1), lambda i: (i, 0)),
                  pl.BlockSpec((1, _D), lambda i: (0, 0))],
        out_specs=pl.BlockSpec((_BM, _D), lambda i: (i, 0)),
        out_shape=jax.ShapeDtypeStruct((_NP, _D), jnp.float32),
    )(parts, parts, hp, dis, b)


def kernel(node_feat, edge_index, W1, b1, gamma, beta, W2, b2):
    src = edge_index[0]
    dst = edge_index[1]
    pad = _N + (jnp.arange(_EP - _E, dtype=jnp.int32) % (_NP - _N))
    src_rows = jnp.concatenate([src, pad]).reshape(_EROWS, _CHUNK)
    dst_rows = jnp.concatenate([dst, pad]).reshape(_EROWS, _CHUNK)
    x_pad = jnp.zeros((_NP, _D), jnp.float32).at[:_N].set(node_feat)
    b1r = b1.reshape(1, _D)
    b2r = b2.reshape(1, _D)
    gr = gamma.reshape(1, _D)
    ber = beta.reshape(1, _D)

    degp = _sc_degree(dst_rows)
    dis = _tc_deg_reduce(degp).reshape(_NP, 1)
    h1p = _tc_matmul_scale(x_pad, W1, dis)
    parts1 = _sc_aggregate(h1p, src_rows, dst_rows)
    conv1, stats = _tc_combine_stats(parts1, h1p, dis, b1r)
    h2p = _tc_bn_matmul(conv1, stats, gr, ber, W2, dis)
    parts2 = _sc_aggregate(h2p, src_rows, dst_rows)
    out = _tc_final(parts2, h2p, dis, b2r)
    return out[:_N]
